# Initial kernel scaffold; baseline (speedup 1.0000x reference)
import jax
import jax.numpy as jnp
from jax.experimental import pallas as pl


def kernel(mask_feature, feature, edge_index, edge_type, W_in, b_in, w1, root1, b1, w2, root2, b2, W_out, b_out):
    raise NotImplementedError("write your pallas kernel here")


# speedup vs baseline: 7.0914x; 7.0914x over previous
import functools

import jax
import jax.numpy as jnp
from jax import lax
from jax.experimental import pallas as pl
from jax.experimental.pallas import tpu as pltpu
from jax.experimental.pallas import tpu_sc as plsc

N = 50000
E = 800000
EMB = 128
HID = 64
OUT = 3
R = 2
NEG_SLOPE = 0.01

NC = 2
NS = 16
CH = 8

ROWS = 6400
EP = ROWS * 128
GARBAGE = 2 * N
ACC = 100096
APT = ACC // NS

f32 = jnp.float32
i32 = jnp.int32


def _fill_const(ref, nrows, val):
    def body(i, _):
        ref[i] = jnp.full((16,), val, f32)
        return 0
    lax.fori_loop(0, nrows, body, 0)


def _zero_acc(acc, zbuf, s):
    base = s * APT
    for k in range(12):
        pltpu.sync_copy(zbuf.at[pl.ds(0, 512)], acc.at[pl.ds(base + k * 512, 512)])
    pltpu.sync_copy(zbuf.at[pl.ds(0, 112)], acc.at[pl.ds(base + 6144, 112)])


def _edge_chunk(table, src_idx, dst_idx, src_base, dst_base,
                idxv, didxv, valsv, acc, sem):
    pltpu.sync_copy(src_idx.at[pl.ds(src_base, CH)], idxv)
    pltpu.sync_copy(dst_idx.at[pl.ds(dst_base, CH)], didxv)
    copies = [pltpu.async_copy(table.at[idxv.at[j]], valsv.at[j], sem)
              for j in range(CH)]
    for d in copies:
        d.wait()
    for j in range(CH):
        pltpu.sync_copy(valsv.at[j], acc.at[didxv.at[j]], add=True)


def _r1_body(x8, srcg, gdst, sums_out, cnt_out,
             acc, idxv, didxv, valsv, onesv, zbuf, sem):
    c = lax.axis_index("c")
    s = lax.axis_index("s")
    _fill_const(zbuf, 512, 0.0)
    _fill_const(onesv, 128, 1.0)
    wbase = s * APT

    for p in range(4):
        g = c * 4 + p
        _zero_acc(acc, zbuf, s)
        plsc.subcore_barrier()

        def chunk(n, _):
            rb = s * (ROWS // NS) + n * CH
            _edge_chunk(x8, srcg, gdst, g * ROWS + rb, rb,
                        idxv, didxv, valsv, acc, sem)
            return 0
        lax.fori_loop(0, ROWS // NS // CH, chunk, 0)
        plsc.subcore_barrier()
        pltpu.sync_copy(acc.at[pl.ds(wbase, APT)],
                        sums_out.at[pl.ds(g * ACC + wbase, APT)])
        plsc.subcore_barrier()

    _zero_acc(acc, zbuf, s)
    plsc.subcore_barrier()

    def cchunk(n, _):
        rb = c * (ROWS // 2) + s * (ROWS // 2 // NS) + n * CH
        pltpu.sync_copy(gdst.at[pl.ds(rb, CH)], didxv)
        for j in range(CH):
            pltpu.sync_copy(onesv, acc.at[didxv.at[j]], add=True)
        return 0
    lax.fori_loop(0, ROWS // 2 // NS // CH, cchunk, 0)
    plsc.subcore_barrier()
    pltpu.sync_copy(acc.at[pl.ds(wbase, APT)],
                    cnt_out.at[pl.ds(c * ACC + wbase, APT)])


def _r2_body(ytab, gsrc, gdst, t_out, acc, idxv, didxv, valsv, zbuf, sem):
    c = lax.axis_index("c")
    s = lax.axis_index("s")
    _fill_const(zbuf, 512, 0.0)
    wbase = s * APT
    _zero_acc(acc, zbuf, s)
    plsc.subcore_barrier()

    def chunk(n, _):
        rb = c * (ROWS // 2) + s * (ROWS // 2 // NS) + n * CH
        _edge_chunk(ytab, gsrc, gdst, rb, rb, idxv, didxv, valsv, acc, sem)
        return 0
    lax.fori_loop(0, ROWS // 2 // NS // CH, chunk, 0)
    plsc.subcore_barrier()
    pltpu.sync_copy(acc.at[pl.ds(wbase, APT)],
                    t_out.at[pl.ds(c * ACC + wbase, APT)])


_sc_mesh = plsc.VectorSubcoreMesh(core_axis_name="c", subcore_axis_name="s")
_sc_params = pltpu.CompilerParams(use_tc_tiling_on_sc=False)

_r1_call = pl.kernel(
    _r1_body,
    compiler_params=_sc_params,
    out_type=(jax.ShapeDtypeStruct((8 * ACC, 16), f32),
              jax.ShapeDtypeStruct((2 * ACC, 16), f32)),
    mesh=_sc_mesh,
    scratch_types=[
        pltpu.VMEM_SHARED((ACC, 16), f32),
        pltpu.VMEM((CH, 128), i32),
        pltpu.VMEM((CH, 128), i32),
        pltpu.VMEM((CH, 128, 16), f32),
        pltpu.VMEM((128, 16), f32),
        pltpu.VMEM((512, 16), f32),
        pltpu.SemaphoreType.DMA,
    ],
)

_r2_call = pl.kernel(
    _r2_body,
    compiler_params=_sc_params,
    out_type=jax.ShapeDtypeStruct((2 * ACC, 16), f32),
    mesh=_sc_mesh,
    scratch_types=[
        pltpu.VMEM_SHARED((ACC, 16), f32),
        pltpu.VMEM((CH, 128), i32),
        pltpu.VMEM((CH, 128), i32),
        pltpu.VMEM((CH, 128, 16), f32),
        pltpu.VMEM((512, 16), f32),
        pltpu.SemaphoreType.DMA,
    ],
)


BN = 2000
GRID = N // BN


def _lrelu(x):
    return jnp.where(x >= 0, x, NEG_SLOPE * x)


def _tc1_body(m_ref, fd_ref, w_ref, b_ref, o_ref):
    m = m_ref[...]
    d = fd_ref[...]
    w = w_ref[...]
    b = b_ref[...]
    a = _lrelu(jnp.dot(m, w, preferred_element_type=f32) + b)
    k = _lrelu(jnp.dot(d - m, w, preferred_element_type=f32) + b)
    o_ref[...] = jnp.concatenate([a, k], axis=1)


def _tc2_body(x_ref, s_ref, cnt_ref, bdr1_ref, bdw1_ref, p_ref, b1_ref,
              h_ref, y_ref):
    x = x_ref[...]
    sS = s_ref[...]
    cnt = cnt_ref[...]
    csum = jnp.maximum(cnt[0] + cnt[1], 1.0)
    m0 = sS[0] / csum[0]
    m1 = sS[1] / csum[1]
    h = (jnp.dot(x, bdr1_ref[...], preferred_element_type=f32)
         + b1_ref[...]
         + jnp.dot(m0, bdw1_ref[0], preferred_element_type=f32)
         + jnp.dot(m1, bdw1_ref[1], preferred_element_type=f32))
    h_ref[...] = h
    y0 = jnp.dot(h, p_ref[0], preferred_element_type=f32)
    y1 = jnp.dot(h, p_ref[1], preferred_element_type=f32)
    y_ref[...] = jnp.stack([y0, y1], axis=0)


def _tc3_body(h_ref, t_ref, cnt_ref, q_ref, bq_ref, o_ref):
    h = h_ref[...]
    t = t_ref[...]
    cnt = cnt_ref[...]
    csum = jnp.maximum(cnt[0] + cnt[1], 1.0)
    tsum = t[0] + t[1]
    agg = tsum[0] / csum[0] + tsum[1] / csum[1]
    o16 = jnp.dot(h, q_ref[...], preferred_element_type=f32) + bq_ref[...] + agg
    a = lax.slice(o16, (0, 0), (BN, OUT))
    k = lax.slice(o16, (0, OUT), (BN, 2 * OUT))
    o_ref[...] = a * k


def _block_diag(a):
    r, c = a.shape
    z = jnp.zeros((2 * r, 2 * c), f32)
    return z.at[:r, :c].set(a).at[r:, c:].set(a)


def _pad16(a):
    return jnp.pad(a, ((0, 0), (0, 16 - a.shape[1])))


@jax.jit
def _impl(mask_feature, feature, edge_index, edge_type,
          W_in, b_in, w1, root1, b1, w2, root2, b2, W_out, b_out):
    src = edge_index[0]
    dst = edge_index[1]
    npad = EP - E
    src_p = jnp.concatenate([src, jnp.zeros((npad,), i32)])
    gdst_p = jnp.concatenate([edge_type * N + dst,
                              jnp.full((npad,), GARBAGE, i32)])
    gsrc_p = jnp.concatenate([edge_type * N + src, jnp.zeros((npad,), i32)])
    srcg = (src_p.reshape(1, ROWS, 128)
            + (jnp.arange(8, dtype=i32) * N)[:, None, None]).reshape(8 * ROWS, 128)
    gdst2 = gdst_p.reshape(ROWS, 128)
    gsrc2 = gsrc_p.reshape(ROWS, 128)

    x_fused = pl.pallas_call(
        _tc1_body,
        grid=(GRID,),
        in_specs=[
            pl.BlockSpec((BN, EMB), lambda i: (i, 0)),
            pl.BlockSpec((BN, EMB), lambda i: (i, 0)),
            pl.BlockSpec((EMB, HID), lambda i: (0, 0)),
            pl.BlockSpec((1, HID), lambda i: (0, 0)),
        ],
        out_specs=pl.BlockSpec((BN, 2 * HID), lambda i: (i, 0)),
        out_shape=jax.ShapeDtypeStruct((N, 2 * HID), f32),
    )(mask_feature, feature, W_in, b_in[None, :])

    x8 = x_fused.reshape(N, 8, 16).transpose(1, 0, 2).reshape(8 * N, 16)
    sums, cnts = _r1_call(x8, srcg, gdst2)
    sS = (sums.reshape(8, ACC, 16)[:, :2 * N, :]
          .reshape(8, 2, N, 16).transpose(1, 2, 0, 3).reshape(2, N, 2 * HID))
    cnt4 = cnts.reshape(2, ACC, 16)[:, :2 * N, :1].reshape(2, 2, N, 1)

    bdr1 = _block_diag(root1)
    bdw1 = jnp.stack([_block_diag(w1[0]), _block_diag(w1[1])])
    p_fold = jnp.stack([_pad16(_block_diag(w2[0] @ W_out)),
                        _pad16(_block_diag(w2[1] @ W_out))])
    h1, y2 = pl.pallas_call(
        _tc2_body,
        grid=(GRID,),
        in_specs=[
            pl.BlockSpec((BN, 2 * HID), lambda i: (i, 0)),
            pl.BlockSpec((2, BN, 2 * HID), lambda i: (0, i, 0)),
            pl.BlockSpec((2, 2, BN, 1), lambda i: (0, 0, i, 0)),
            pl.BlockSpec((2 * HID, 2 * HID), lambda i: (0, 0)),
            pl.BlockSpec((2, 2 * HID, 2 * HID), lambda i: (0, 0, 0)),
            pl.BlockSpec((2, 2 * HID, 16), lambda i: (0, 0, 0)),
            pl.BlockSpec((1, 2 * HID), lambda i: (0, 0)),
        ],
        out_specs=[
            pl.BlockSpec((BN, 2 * HID), lambda i: (i, 0)),
            pl.BlockSpec((2, BN, 16), lambda i: (0, i, 0)),
        ],
        out_shape=[
            jax.ShapeDtypeStruct((N, 2 * HID), f32),
            jax.ShapeDtypeStruct((2, N, 16), f32),
        ],
    )(x_fused, sS, cnt4, bdr1, bdw1, p_fold, jnp.tile(b1, 2)[None, :])

    ytab = y2.reshape(2 * N, 16)
    t_parts = _r2_call(ytab, gsrc2, gdst2)
    t4 = t_parts.reshape(2, ACC, 16)[:, :2 * N, :].reshape(2, 2, N, 16)

    q = _pad16(_block_diag(root2 @ W_out))
    bfin = b2 @ W_out + b_out
    bq = jnp.concatenate([bfin, bfin, jnp.zeros((16 - 2 * OUT,), f32)])[None, :]
    out = pl.pallas_call(
        _tc3_body,
        grid=(GRID,),
        in_specs=[
            pl.BlockSpec((BN, 2 * HID), lambda i: (i, 0)),
            pl.BlockSpec((2, 2, BN, 16), lambda i: (0, 0, i, 0)),
            pl.BlockSpec((2, 2, BN, 1), lambda i: (0, 0, i, 0)),
            pl.BlockSpec((2 * HID, 16), lambda i: (0, 0)),
            pl.BlockSpec((1, 16), lambda i: (0, 0)),
        ],
        out_specs=pl.BlockSpec((BN, OUT), lambda i: (i, 0)),
        out_shape=jax.ShapeDtypeStruct((N, OUT), f32),
    )(h1, t4, cnt4, q, bq)
    return out


def kernel(mask_feature, feature, edge_index, edge_type,
           W_in, b_in, w1, root1, b1, w2, root2, b2, W_out, b_out):
    return _impl(mask_feature, feature, edge_index, edge_type,
                 W_in, b_in, w1, root1, b1, w2, root2, b2, W_out, b_out)


# speedup vs baseline: 8.0538x; 1.1357x over previous
import functools

import jax
import jax.numpy as jnp
from jax import lax
from jax.experimental import pallas as pl
from jax.experimental.pallas import tpu as pltpu
from jax.experimental.pallas import tpu_sc as plsc

N = 50000
E = 800000
EMB = 128
HID = 64
OUT = 3
R = 2
NEG_SLOPE = 0.01

NC = 2
NS = 16
CH = 4

ROWS = 6400
EP = ROWS * 128
GARBAGE = 2 * N
ACC = 100096
APT = ACC // NS

f32 = jnp.float32
i32 = jnp.int32


def _fill_const(ref, nrows, val):
    def body(i, _):
        ref[i] = jnp.full((16,), val, f32)
        return 0
    lax.fori_loop(0, nrows, body, 0)


def _drain(table, buf, sem):
    pltpu.make_async_copy(table.at[pl.ds(0, CH * 128)], buf, sem).wait()


def _prefetch(table, src_idx, dst_idx, src_base, dst_base, chunk_id,
              idxv, didxv, valsv, sg, ss, first_round):
    @pl.when(jnp.logical_not(first_round))
    def _():
        _drain(table, valsv, ss)
    pltpu.sync_copy(src_idx.at[pl.ds(src_base, CH)], idxv)
    pltpu.sync_copy(dst_idx.at[pl.ds(dst_base, CH)], didxv)
    for j in range(CH):
        pltpu.async_copy(table.at[idxv.at[j]], valsv.at[pl.ds(j * 128, 128)], sg)


def _commit(table, didxv, valsv, acc, sg, ss):
    _drain(table, valsv, sg)
    for j in range(CH):
        pltpu.async_copy(valsv.at[pl.ds(j * 128, 128)], acc.at[didxv.at[j]],
                         ss, add=True)


def _run_edge_pass(table, src_idx, dst_idx, sbase_fn, dbase_fn, nch, acc,
                   idx0, didx0, vals0, idx1, didx1, vals1,
                   sg0, sg1, ss0, ss1):
    _prefetch(table, src_idx, dst_idx, sbase_fn(0), dbase_fn(0), 0,
              idx0, didx0, vals0, sg0, ss0, jnp.bool_(True))

    def step(k, _):
        c0 = 2 * k
        c1 = c0 + 1
        _prefetch(table, src_idx, dst_idx, sbase_fn(c1), dbase_fn(c1), c1,
                  idx1, didx1, vals1, sg1, ss1, k == 0)
        _commit(table, didx0, vals0, acc, sg0, ss0)

        @pl.when(c0 + 2 < nch)
        def _():
            _prefetch(table, src_idx, dst_idx, sbase_fn(c0 + 2),
                      dbase_fn(c0 + 2), c0 + 2,
                      idx0, didx0, vals0, sg0, ss0, jnp.bool_(False))
        _commit(table, didx1, vals1, acc, sg1, ss1)
        return 0

    lax.fori_loop(0, nch // 2, step, 0)
    _drain(table, vals0, ss0)
    _drain(table, vals1, ss1)


def _r1_body(x8, srcg, gdst, zrows, sums_out, cnt_out,
             acc, idx0, didx0, vals0, idx1, didx1, vals1, onesv,
             sg0, sg1, ss0, ss1):
    c = lax.axis_index("c")
    s = lax.axis_index("s")
    _fill_const(onesv, 128, 1.0)
    wbase = s * APT

    for p in range(4):
        g = c * 4 + p
        pltpu.sync_copy(zrows, acc.at[pl.ds(wbase, APT)])
        plsc.subcore_barrier()
        rpt = ROWS // NS
        _run_edge_pass(
            x8, srcg, gdst,
            lambda n, g=g, rpt=rpt: g * ROWS + s * rpt + n * CH,
            lambda n, rpt=rpt: s * rpt + n * CH,
            rpt // CH, acc,
            idx0, didx0, vals0, idx1, didx1, vals1, sg0, sg1, ss0, ss1)
        plsc.subcore_barrier()
        pltpu.sync_copy(acc.at[pl.ds(wbase, APT)],
                        sums_out.at[pl.ds(g * ACC + wbase, APT)])
        plsc.subcore_barrier()

    pltpu.sync_copy(zrows, acc.at[pl.ds(wbase, APT)])
    plsc.subcore_barrier()
    hpt = ROWS // 2 // NS

    def cstep(k, _):
        c0 = 2 * k
        for cc, didx, ss, vals in ((c0, didx0, ss0, vals0),
                                   (c0 + 1, didx1, ss1, vals1)):
            rb = c * (ROWS // 2) + s * hpt + cc * CH

            @pl.when(jnp.logical_not(k == 0))
            def _():
                _drain(x8, vals, ss)
            pltpu.sync_copy(gdst.at[pl.ds(rb, CH)], didx)
            for j in range(CH):
                pltpu.async_copy(onesv, acc.at[didx.at[j]], ss, add=True)
        return 0

    lax.fori_loop(0, hpt // CH // 2, cstep, 0)
    _drain(x8, vals0, ss0)
    _drain(x8, vals1, ss1)
    plsc.subcore_barrier()
    pltpu.sync_copy(acc.at[pl.ds(wbase, APT)],
                    cnt_out.at[pl.ds(c * ACC + wbase, APT)])


def _r2_body(ytab, gsrc, gdst, zrows, t_out,
             acc, idx0, didx0, vals0, idx1, didx1, vals1,
             sg0, sg1, ss0, ss1):
    c = lax.axis_index("c")
    s = lax.axis_index("s")
    wbase = s * APT
    pltpu.sync_copy(zrows, acc.at[pl.ds(wbase, APT)])
    plsc.subcore_barrier()
    hpt = ROWS // 2 // NS

    def base(n):
        return c * (ROWS // 2) + s * hpt + n * CH

    _run_edge_pass(ytab, gsrc, gdst, base, base, hpt // CH, acc,
                   idx0, didx0, vals0, idx1, didx1, vals1,
                   sg0, sg1, ss0, ss1)
    plsc.subcore_barrier()
    pltpu.sync_copy(acc.at[pl.ds(wbase, APT)],
                    t_out.at[pl.ds(c * ACC + wbase, APT)])


_sc_mesh = plsc.VectorSubcoreMesh(core_axis_name="c", subcore_axis_name="s")
_sc_params = pltpu.CompilerParams(use_tc_tiling_on_sc=False)

_pipe_scratch = [
    pltpu.VMEM((CH, 128), i32),
    pltpu.VMEM((CH, 128), i32),
    pltpu.VMEM((CH * 128, 16), f32),
    pltpu.VMEM((CH, 128), i32),
    pltpu.VMEM((CH, 128), i32),
    pltpu.VMEM((CH * 128, 16), f32),
]
_pipe_sems = [pltpu.SemaphoreType.DMA] * 4

_r1_call = pl.kernel(
    _r1_body,
    compiler_params=_sc_params,
    out_type=(jax.ShapeDtypeStruct((8 * ACC, 16), f32),
              jax.ShapeDtypeStruct((2 * ACC, 16), f32)),
    mesh=_sc_mesh,
    scratch_types=[pltpu.VMEM_SHARED((ACC, 16), f32)] + _pipe_scratch
    + [pltpu.VMEM((128, 16), f32)] + _pipe_sems,
)

_r2_call = pl.kernel(
    _r2_body,
    compiler_params=_sc_params,
    out_type=jax.ShapeDtypeStruct((2 * ACC, 16), f32),
    mesh=_sc_mesh,
    scratch_types=[pltpu.VMEM_SHARED((ACC, 16), f32)] + _pipe_scratch
    + _pipe_sems,
)


BN = 2000
GRID = N // BN


def _lrelu(x):
    return jnp.where(x >= 0, x, NEG_SLOPE * x)


def _tc1_body(m_ref, fd_ref, w_ref, b_ref, o_ref):
    m = m_ref[...]
    d = fd_ref[...]
    w = w_ref[...]
    b = b_ref[...]
    a = _lrelu(jnp.dot(m, w, preferred_element_type=f32) + b)
    k = _lrelu(jnp.dot(d - m, w, preferred_element_type=f32) + b)
    o_ref[...] = jnp.concatenate([a, k], axis=1)


def _tc2_body(x_ref, s_ref, cnt_ref, bdr1_ref, bdw1_ref, p_ref, b1_ref,
              h_ref, y_ref):
    x = x_ref[...]
    sS = s_ref[...]
    cnt = cnt_ref[...]
    csum = jnp.maximum(cnt[0] + cnt[1], 1.0)
    m0 = sS[0] / csum[0]
    m1 = sS[1] / csum[1]
    h = (jnp.dot(x, bdr1_ref[...], preferred_element_type=f32)
         + b1_ref[...]
         + jnp.dot(m0, bdw1_ref[0], preferred_element_type=f32)
         + jnp.dot(m1, bdw1_ref[1], preferred_element_type=f32))
    h_ref[...] = h
    y0 = jnp.dot(h, p_ref[0], preferred_element_type=f32)
    y1 = jnp.dot(h, p_ref[1], preferred_element_type=f32)
    y_ref[...] = jnp.stack([y0, y1], axis=0)


def _tc3_body(h_ref, t_ref, cnt_ref, q_ref, bq_ref, o_ref):
    h = h_ref[...]
    t = t_ref[...]
    cnt = cnt_ref[...]
    csum = jnp.maximum(cnt[0] + cnt[1], 1.0)
    tsum = t[0] + t[1]
    agg = tsum[0] / csum[0] + tsum[1] / csum[1]
    o16 = jnp.dot(h, q_ref[...], preferred_element_type=f32) + bq_ref[...] + agg
    a = lax.slice(o16, (0, 0), (BN, OUT))
    k = lax.slice(o16, (0, OUT), (BN, 2 * OUT))
    o_ref[...] = a * k


def _block_diag(a):
    r, c = a.shape
    z = jnp.zeros((2 * r, 2 * c), f32)
    return z.at[:r, :c].set(a).at[r:, c:].set(a)


def _pad16(a):
    return jnp.pad(a, ((0, 0), (0, 16 - a.shape[1])))


@jax.jit
def _impl(mask_feature, feature, edge_index, edge_type,
          W_in, b_in, w1, root1, b1, w2, root2, b2, W_out, b_out):
    src = edge_index[0]
    dst = edge_index[1]
    npad = EP - E
    src_p = jnp.concatenate([src, jnp.zeros((npad,), i32)])
    gdst_p = jnp.concatenate([edge_type * N + dst,
                              jnp.full((npad,), GARBAGE, i32)])
    gsrc_p = jnp.concatenate([edge_type * N + src, jnp.zeros((npad,), i32)])
    srcg = (src_p.reshape(1, ROWS, 128)
            + (jnp.arange(8, dtype=i32) * N)[:, None, None]).reshape(8 * ROWS, 128)
    gdst2 = gdst_p.reshape(ROWS, 128)
    gsrc2 = gsrc_p.reshape(ROWS, 128)

    x_fused = pl.pallas_call(
        _tc1_body,
        grid=(GRID,),
        in_specs=[
            pl.BlockSpec((BN, EMB), lambda i: (i, 0)),
            pl.BlockSpec((BN, EMB), lambda i: (i, 0)),
            pl.BlockSpec((EMB, HID), lambda i: (0, 0)),
            pl.BlockSpec((1, HID), lambda i: (0, 0)),
        ],
        out_specs=pl.BlockSpec((BN, 2 * HID), lambda i: (i, 0)),
        out_shape=jax.ShapeDtypeStruct((N, 2 * HID), f32),
    )(mask_feature, feature, W_in, b_in[None, :])

    x8 = x_fused.reshape(N, 8, 16).transpose(1, 0, 2).reshape(8 * N, 16)
    zrows = jnp.zeros((APT, 16), f32)
    sums, cnts = _r1_call(x8, srcg, gdst2, zrows)
    sS = (sums.reshape(8, ACC, 16)[:, :2 * N, :]
          .reshape(8, 2, N, 16).transpose(1, 2, 0, 3).reshape(2, N, 2 * HID))
    cnt4 = cnts.reshape(2, ACC, 16)[:, :2 * N, :1].reshape(2, 2, N, 1)

    bdr1 = _block_diag(root1)
    bdw1 = jnp.stack([_block_diag(w1[0]), _block_diag(w1[1])])
    p_fold = jnp.stack([_pad16(_block_diag(w2[0] @ W_out)),
                        _pad16(_block_diag(w2[1] @ W_out))])
    h1, y2 = pl.pallas_call(
        _tc2_body,
        grid=(GRID,),
        in_specs=[
            pl.BlockSpec((BN, 2 * HID), lambda i: (i, 0)),
            pl.BlockSpec((2, BN, 2 * HID), lambda i: (0, i, 0)),
            pl.BlockSpec((2, 2, BN, 1), lambda i: (0, 0, i, 0)),
            pl.BlockSpec((2 * HID, 2 * HID), lambda i: (0, 0)),
            pl.BlockSpec((2, 2 * HID, 2 * HID), lambda i: (0, 0, 0)),
            pl.BlockSpec((2, 2 * HID, 16), lambda i: (0, 0, 0)),
            pl.BlockSpec((1, 2 * HID), lambda i: (0, 0)),
        ],
        out_specs=[
            pl.BlockSpec((BN, 2 * HID), lambda i: (i, 0)),
            pl.BlockSpec((2, BN, 16), lambda i: (0, i, 0)),
        ],
        out_shape=[
            jax.ShapeDtypeStruct((N, 2 * HID), f32),
            jax.ShapeDtypeStruct((2, N, 16), f32),
        ],
    )(x_fused, sS, cnt4, bdr1, bdw1, p_fold, jnp.tile(b1, 2)[None, :])

    ytab = y2.reshape(2 * N, 16)
    t_parts = _r2_call(ytab, gsrc2, gdst2, zrows)
    t4 = t_parts.reshape(2, ACC, 16)[:, :2 * N, :].reshape(2, 2, N, 16)

    q = _pad16(_block_diag(root2 @ W_out))
    bfin = b2 @ W_out + b_out
    bq = jnp.concatenate([bfin, bfin, jnp.zeros((16 - 2 * OUT,), f32)])[None, :]
    out = pl.pallas_call(
        _tc3_body,
        grid=(GRID,),
        in_specs=[
            pl.BlockSpec((BN, 2 * HID), lambda i: (i, 0)),
            pl.BlockSpec((2, 2, BN, 16), lambda i: (0, 0, i, 0)),
            pl.BlockSpec((2, 2, BN, 1), lambda i: (0, 0, i, 0)),
            pl.BlockSpec((2 * HID, 16), lambda i: (0, 0)),
            pl.BlockSpec((1, 16), lambda i: (0, 0)),
        ],
        out_specs=pl.BlockSpec((BN, OUT), lambda i: (i, 0)),
        out_shape=jax.ShapeDtypeStruct((N, OUT), f32),
    )(h1, t4, cnt4, q, bq)
    return out


def kernel(mask_feature, feature, edge_index, edge_type,
           W_in, b_in, w1, root1, b1, w2, root2, b2, W_out, b_out):
    return _impl(mask_feature, feature, edge_index, edge_type,
                 W_in, b_in, w1, root1, b1, w2, root2, b2, W_out, b_out)


# speedup vs baseline: 8.2237x; 1.0211x over previous
import functools

import jax
import jax.numpy as jnp
from jax import lax
from jax.experimental import pallas as pl
from jax.experimental.pallas import tpu as pltpu
from jax.experimental.pallas import tpu_sc as plsc

N = 50000
E = 800000
EMB = 128
HID = 64
OUT = 3
R = 2
NEG_SLOPE = 0.01

NC = 2
NS = 16
CH = 4

ROWS = 6400
EP = ROWS * 128
GARBAGE = 2 * N
ACC = 100096
APT = ACC // NS

f32 = jnp.float32
i32 = jnp.int32


def _fill_const(ref, nrows, val):
    def body(i, _):
        ref[i] = jnp.full((16,), val, f32)
        return 0
    lax.fori_loop(0, nrows, body, 0)


def _drain(table, buf, sem):
    pltpu.make_async_copy(table.at[pl.ds(0, CH * 128)], buf, sem).wait()


def _prefetch(table, src_idx, dst_idx, src_base, dst_base, src_off,
              idxv, didxv, valsv, sg, ss, first_round):
    @pl.when(jnp.logical_not(first_round))
    def _():
        _drain(table, valsv, ss)
    pltpu.sync_copy(src_idx.at[pl.ds(src_base, CH)], idxv)
    pltpu.sync_copy(dst_idx.at[pl.ds(dst_base, CH)], didxv)
    if src_off is not None:
        for jj in range(CH):
            for kk in range(8):
                idxv[jj, pl.ds(kk * 16, 16)] = idxv[jj, pl.ds(kk * 16, 16)] + src_off
    for j in range(CH):
        pltpu.async_copy(table.at[idxv.at[j]], valsv.at[pl.ds(j * 128, 128)], sg)


def _commit(table, didxv, valsv, acc, sg, ss):
    _drain(table, valsv, sg)
    for j in range(CH):
        pltpu.async_copy(valsv.at[pl.ds(j * 128, 128)], acc.at[didxv.at[j]],
                         ss, add=True)


def _run_edge_pass(table, src_idx, dst_idx, sbase_fn, dbase_fn, src_off,
                   nch, acc, idx0, didx0, vals0, idx1, didx1, vals1,
                   sg0, sg1, ss0, ss1):
    _prefetch(table, src_idx, dst_idx, sbase_fn(0), dbase_fn(0), src_off,
              idx0, didx0, vals0, sg0, ss0, jnp.bool_(True))

    def step(k, _):
        c0 = 2 * k
        c1 = c0 + 1
        _prefetch(table, src_idx, dst_idx, sbase_fn(c1), dbase_fn(c1),
                  src_off, idx1, didx1, vals1, sg1, ss1, k == 0)
        _commit(table, didx0, vals0, acc, sg0, ss0)

        @pl.when(c0 + 2 < nch)
        def _():
            _prefetch(table, src_idx, dst_idx, sbase_fn(c0 + 2),
                      dbase_fn(c0 + 2), src_off,
                      idx0, didx0, vals0, sg0, ss0, jnp.bool_(False))
        _commit(table, didx1, vals1, acc, sg1, ss1)
        return 0

    lax.fori_loop(0, nch // 2, step, 0)
    _drain(table, vals0, ss0)
    _drain(table, vals1, ss1)


def _r1_body(x8, src2, gdst, zrows, sums_out, cnt_out,
             acc, idx0, didx0, vals0, idx1, didx1, vals1, onesv,
             sg0, sg1, ss0, ss1):
    c = lax.axis_index("c")
    s = lax.axis_index("s")
    _fill_const(onesv, 128, 1.0)
    wbase = s * APT

    for p in range(4):
        g = c * 4 + p
        pltpu.sync_copy(zrows, acc.at[pl.ds(wbase, APT)])
        plsc.subcore_barrier()
        rpt = ROWS // NS
        _run_edge_pass(
            x8, src2, gdst,
            lambda n, rpt=rpt: s * rpt + n * CH,
            lambda n, rpt=rpt: s * rpt + n * CH,
            g * N, rpt // CH, acc,
            idx0, didx0, vals0, idx1, didx1, vals1, sg0, sg1, ss0, ss1)
        plsc.subcore_barrier()
        pltpu.sync_copy(acc.at[pl.ds(wbase, APT)],
                        sums_out.at[pl.ds(g * ACC + wbase, APT)])
        plsc.subcore_barrier()

    pltpu.sync_copy(zrows, acc.at[pl.ds(wbase, APT)])
    plsc.subcore_barrier()
    hpt = ROWS // 2 // NS

    def cstep(k, _):
        c0 = 2 * k
        for cc, didx, ss, vals in ((c0, didx0, ss0, vals0),
                                   (c0 + 1, didx1, ss1, vals1)):
            rb = c * (ROWS // 2) + s * hpt + cc * CH

            @pl.when(jnp.logical_not(k == 0))
            def _():
                _drain(x8, vals, ss)
            pltpu.sync_copy(gdst.at[pl.ds(rb, CH)], didx)
            for j in range(CH):
                pltpu.async_copy(onesv, acc.at[didx.at[j]], ss, add=True)
        return 0

    lax.fori_loop(0, hpt // CH // 2, cstep, 0)
    _drain(x8, vals0, ss0)
    _drain(x8, vals1, ss1)
    plsc.subcore_barrier()
    pltpu.sync_copy(acc.at[pl.ds(wbase, APT)],
                    cnt_out.at[pl.ds(c * ACC + wbase, APT)])


def _r2_body(ytab, gsrc, gdst, zrows, t_out,
             acc, idx0, didx0, vals0, idx1, didx1, vals1,
             sg0, sg1, ss0, ss1):
    c = lax.axis_index("c")
    s = lax.axis_index("s")
    wbase = s * APT
    pltpu.sync_copy(zrows, acc.at[pl.ds(wbase, APT)])
    plsc.subcore_barrier()
    hpt = ROWS // 2 // NS

    def base(n):
        return c * (ROWS // 2) + s * hpt + n * CH

    _run_edge_pass(ytab, gsrc, gdst, base, base, None, hpt // CH, acc,
                   idx0, didx0, vals0, idx1, didx1, vals1,
                   sg0, sg1, ss0, ss1)
    plsc.subcore_barrier()
    pltpu.sync_copy(acc.at[pl.ds(wbase, APT)],
                    t_out.at[pl.ds(c * ACC + wbase, APT)])


_sc_mesh = plsc.VectorSubcoreMesh(core_axis_name="c", subcore_axis_name="s")
_sc_params = pltpu.CompilerParams(use_tc_tiling_on_sc=False)

_pipe_scratch = [
    pltpu.VMEM((CH, 128), i32),
    pltpu.VMEM((CH, 128), i32),
    pltpu.VMEM((CH * 128, 16), f32),
    pltpu.VMEM((CH, 128), i32),
    pltpu.VMEM((CH, 128), i32),
    pltpu.VMEM((CH * 128, 16), f32),
]
_pipe_sems = [pltpu.SemaphoreType.DMA] * 4

_r1_call = pl.kernel(
    _r1_body,
    compiler_params=_sc_params,
    out_type=(jax.ShapeDtypeStruct((8 * ACC, 16), f32),
              jax.ShapeDtypeStruct((2 * ACC, 16), f32)),
    mesh=_sc_mesh,
    scratch_types=[pltpu.VMEM_SHARED((ACC, 16), f32)] + _pipe_scratch
    + [pltpu.VMEM((128, 16), f32)] + _pipe_sems,
)

_r2_call = pl.kernel(
    _r2_body,
    compiler_params=_sc_params,
    out_type=jax.ShapeDtypeStruct((2 * ACC, 16), f32),
    mesh=_sc_mesh,
    scratch_types=[pltpu.VMEM_SHARED((ACC, 16), f32)] + _pipe_scratch
    + _pipe_sems,
)


BN = 2000
GRID = N // BN


def _lrelu(x):
    return jnp.where(x >= 0, x, NEG_SLOPE * x)


def _tc1_body(m_ref, fd_ref, w_ref, b_ref, o_ref, o8_ref):
    m = m_ref[...]
    d = fd_ref[...]
    w = w_ref[...]
    b = b_ref[...]
    a = _lrelu(jnp.dot(m, w, preferred_element_type=f32, precision=jax.lax.Precision.HIGHEST) + b)
    k = _lrelu(jnp.dot(d - m, w, preferred_element_type=f32, precision=jax.lax.Precision.HIGHEST) + b)
    full = jnp.concatenate([a, k], axis=1)
    o_ref[...] = full
    for g in range(8):
        o8_ref[g] = lax.slice(full, (0, 16 * g), (BN, 16 * (g + 1)))


def _tc2_body(x_ref, sa_ref, sb_ref, cnt_ref, bdr1_ref, bdw1_ref, p_ref,
              b1_ref, h_ref, y_ref):
    x = x_ref[...]
    sa = sa_ref[...]
    sb = sb_ref[...]
    cnt = cnt_ref[...]
    csum = jnp.maximum(cnt[0] + cnt[1], 1.0)
    ir0 = 1.0 / csum[0]
    ir1 = 1.0 / csum[1]
    agg0 = sum(jnp.dot(sa[g] * ir0, bdw1_ref[0, 16 * g:16 * (g + 1), :],
                       preferred_element_type=f32, precision=jax.lax.Precision.HIGHEST) for g in range(8))
    agg1 = sum(jnp.dot(sb[g] * ir1, bdw1_ref[1, 16 * g:16 * (g + 1), :],
                       preferred_element_type=f32, precision=jax.lax.Precision.HIGHEST) for g in range(8))
    h = (jnp.dot(x, bdr1_ref[...], preferred_element_type=f32, precision=jax.lax.Precision.HIGHEST)
         + b1_ref[...] + agg0 + agg1)
    h_ref[...] = h
    y0 = jnp.dot(h, p_ref[0], preferred_element_type=f32, precision=jax.lax.Precision.HIGHEST)
    y1 = jnp.dot(h, p_ref[1], preferred_element_type=f32, precision=jax.lax.Precision.HIGHEST)
    y_ref[...] = jnp.stack([y0, y1], axis=0)


def _tc3_body(h_ref, t_ref, cnt_ref, q_ref, bq_ref, o_ref):
    h = h_ref[...]
    t = t_ref[...]
    cnt = cnt_ref[...]
    csum = jnp.maximum(cnt[0] + cnt[1], 1.0)
    tsum = t[0] + t[1]
    agg = tsum[0] / csum[0] + tsum[1] / csum[1]
    o16 = jnp.dot(h, q_ref[...], preferred_element_type=f32, precision=jax.lax.Precision.HIGHEST) + bq_ref[...] + agg
    a = lax.slice(o16, (0, 0), (BN, OUT))
    k = lax.slice(o16, (0, OUT), (BN, 2 * OUT))
    o_ref[...] = a * k


def _block_diag(a):
    r, c = a.shape
    z = jnp.zeros((2 * r, 2 * c), f32)
    return z.at[:r, :c].set(a).at[r:, c:].set(a)


def _pad16(a):
    return jnp.pad(a, ((0, 0), (0, 16 - a.shape[1])))


@jax.jit
def _impl(mask_feature, feature, edge_index, edge_type,
          W_in, b_in, w1, root1, b1, w2, root2, b2, W_out, b_out):
    src = edge_index[0]
    dst = edge_index[1]
    npad = EP - E
    src_p = jnp.concatenate([src, jnp.zeros((npad,), i32)])
    gdst_p = jnp.concatenate([edge_type * N + dst,
                              jnp.full((npad,), GARBAGE, i32)])
    gsrc_p = jnp.concatenate([edge_type * N + src, jnp.zeros((npad,), i32)])
    src2 = src_p.reshape(ROWS, 128)
    gdst2 = gdst_p.reshape(ROWS, 128)
    gsrc2 = gsrc_p.reshape(ROWS, 128)

    x_fused, x8t = pl.pallas_call(
        _tc1_body,
        grid=(GRID,),
        in_specs=[
            pl.BlockSpec((BN, EMB), lambda i: (i, 0)),
            pl.BlockSpec((BN, EMB), lambda i: (i, 0)),
            pl.BlockSpec((EMB, HID), lambda i: (0, 0)),
            pl.BlockSpec((1, HID), lambda i: (0, 0)),
        ],
        out_specs=[
            pl.BlockSpec((BN, 2 * HID), lambda i: (i, 0)),
            pl.BlockSpec((8, BN, 16), lambda i: (0, i, 0)),
        ],
        out_shape=[
            jax.ShapeDtypeStruct((N, 2 * HID), f32),
            jax.ShapeDtypeStruct((8, N, 16), f32),
        ],
    )(mask_feature, feature, W_in, b_in[None, :])

    x8 = x8t.reshape(8 * N, 16)
    zrows = jnp.zeros((APT, 16), f32)
    sums, cnts = _r1_call(x8, src2, gdst2, zrows)
    sums3 = sums.reshape(8, ACC, 16)
    cnt4 = cnts.reshape(2, ACC, 16)[:, :2 * N, :1].reshape(2, 2, N, 1)

    bdr1 = _block_diag(root1)
    bdw1 = jnp.stack([_block_diag(w1[0]), _block_diag(w1[1])])
    p_fold = jnp.stack([_pad16(_block_diag(w2[0] @ W_out)),
                        _pad16(_block_diag(w2[1] @ W_out))])
    h1, y2 = pl.pallas_call(
        _tc2_body,
        grid=(GRID,),
        in_specs=[
            pl.BlockSpec((BN, 2 * HID), lambda i: (i, 0)),
            pl.BlockSpec((8, BN, 16), lambda i: (0, i, 0)),
            pl.BlockSpec((8, BN, 16), lambda i: (0, N // BN + i, 0)),
            pl.BlockSpec((2, 2, BN, 1), lambda i: (0, 0, i, 0)),
            pl.BlockSpec((2 * HID, 2 * HID), lambda i: (0, 0)),
            pl.BlockSpec((2, 2 * HID, 2 * HID), lambda i: (0, 0, 0)),
            pl.BlockSpec((2, 2 * HID, 16), lambda i: (0, 0, 0)),
            pl.BlockSpec((1, 2 * HID), lambda i: (0, 0)),
        ],
        out_specs=[
            pl.BlockSpec((BN, 2 * HID), lambda i: (i, 0)),
            pl.BlockSpec((2, BN, 16), lambda i: (0, i, 0)),
        ],
        out_shape=[
            jax.ShapeDtypeStruct((N, 2 * HID), f32),
            jax.ShapeDtypeStruct((2, N, 16), f32),
        ],
    )(x_fused, sums3, sums3, cnt4, bdr1, bdw1, p_fold,
      jnp.tile(b1, 2)[None, :])

    ytab = y2.reshape(2 * N, 16)
    t_parts = _r2_call(ytab, gsrc2, gdst2, zrows)
    t4 = t_parts.reshape(2, ACC, 16)[:, :2 * N, :].reshape(2, 2, N, 16)

    q = _pad16(_block_diag(root2 @ W_out))
    bfin = b2 @ W_out + b_out
    bq = jnp.concatenate([bfin, bfin, jnp.zeros((16 - 2 * OUT,), f32)])[None, :]
    out = pl.pallas_call(
        _tc3_body,
        grid=(GRID,),
        in_specs=[
            pl.BlockSpec((BN, 2 * HID), lambda i: (i, 0)),
            pl.BlockSpec((2, 2, BN, 16), lambda i: (0, 0, i, 0)),
            pl.BlockSpec((2, 2, BN, 1), lambda i: (0, 0, i, 0)),
            pl.BlockSpec((2 * HID, 16), lambda i: (0, 0)),
            pl.BlockSpec((1, 16), lambda i: (0, 0)),
        ],
        out_specs=pl.BlockSpec((BN, OUT), lambda i: (i, 0)),
        out_shape=jax.ShapeDtypeStruct((N, OUT), f32),
    )(h1, t4, cnt4, q, bq)
    return out


def kernel(mask_feature, feature, edge_index, edge_type,
           W_in, b_in, w1, root1, b1, w2, root2, b2, W_out, b_out):
    return _impl(mask_feature, feature, edge_index, edge_type,
                 W_in, b_in, w1, root1, b1, w2, root2, b2, W_out, b_out)


# speedup vs baseline: 9.3305x; 1.1346x over previous
import functools

import jax
import jax.numpy as jnp
from jax import lax
from jax.experimental import pallas as pl
from jax.experimental.pallas import tpu as pltpu
from jax.experimental.pallas import tpu_sc as plsc

N = 50000
E = 800000
EMB = 128
HID = 64
OUT = 3
R = 2
NEG_SLOPE = 0.01

NC = 2
NS = 16
CH = 4

ROWS = 6400
EP = ROWS * 128
GARBAGE = 2 * N
ACC = 100096
APT = ACC // NS

f32 = jnp.float32
i32 = jnp.int32


def _fill_const(ref, nrows, val):
    def body(i, _):
        ref[i] = jnp.full((16,), val, f32)
        return 0
    lax.fori_loop(0, nrows, body, 0)


def _drain(table, buf, sem):
    pltpu.make_async_copy(table.at[pl.ds(0, CH * 128)], buf, sem).wait()


def _prefetch(table, src_idx, dst_idx, src_base, dst_base, src_off,
              idxv, didxv, valsv, sg, ss, first_round):
    @pl.when(jnp.logical_not(first_round))
    def _():
        _drain(table, valsv, ss)
    pltpu.sync_copy(src_idx.at[pl.ds(src_base, CH)], idxv)
    pltpu.sync_copy(dst_idx.at[pl.ds(dst_base, CH)], didxv)
    if src_off is not None:
        for jj in range(CH):
            for kk in range(8):
                idxv[jj, pl.ds(kk * 16, 16)] = idxv[jj, pl.ds(kk * 16, 16)] + src_off
    for j in range(CH):
        pltpu.async_copy(table.at[idxv.at[j]], valsv.at[pl.ds(j * 128, 128)], sg)


def _commit(table, didxv, valsv, acc, sg, ss):
    _drain(table, valsv, sg)
    for j in range(CH):
        pltpu.async_copy(valsv.at[pl.ds(j * 128, 128)], acc.at[didxv.at[j]],
                         ss, add=True)


def _run_edge_pass(table, src_idx, dst_idx, sbase_fn, dbase_fn, src_off,
                   nch, acc, idx0, didx0, vals0, idx1, didx1, vals1,
                   sg0, sg1, ss0, ss1):
    _prefetch(table, src_idx, dst_idx, sbase_fn(0), dbase_fn(0), src_off,
              idx0, didx0, vals0, sg0, ss0, jnp.bool_(True))

    def step(k, _):
        c0 = 2 * k
        c1 = c0 + 1
        _prefetch(table, src_idx, dst_idx, sbase_fn(c1), dbase_fn(c1),
                  src_off, idx1, didx1, vals1, sg1, ss1, k == 0)
        _commit(table, didx0, vals0, acc, sg0, ss0)

        @pl.when(c0 + 2 < nch)
        def _():
            _prefetch(table, src_idx, dst_idx, sbase_fn(c0 + 2),
                      dbase_fn(c0 + 2), src_off,
                      idx0, didx0, vals0, sg0, ss0, jnp.bool_(False))
        _commit(table, didx1, vals1, acc, sg1, ss1)
        return 0

    lax.fori_loop(0, nch // 2, step, 0)
    _drain(table, vals0, ss0)
    _drain(table, vals1, ss1)


def _r1_body(x8, src2, gdst, zrows, sums_out, cnt_out,
             acc, idx0, didx0, vals0, idx1, didx1, vals1, onesv,
             sg0, sg1, ss0, ss1):
    c = lax.axis_index("c")
    s = lax.axis_index("s")
    _fill_const(onesv, 128, 1.0)
    wbase = s * APT

    for p in range(4):
        g = c * 4 + p
        pltpu.sync_copy(zrows, acc.at[pl.ds(wbase, APT)])
        plsc.subcore_barrier()
        rpt = ROWS // NS
        _run_edge_pass(
            x8, src2, gdst,
            lambda n, rpt=rpt: s * rpt + n * CH,
            lambda n, rpt=rpt: s * rpt + n * CH,
            g * N, rpt // CH, acc,
            idx0, didx0, vals0, idx1, didx1, vals1, sg0, sg1, ss0, ss1)
        plsc.subcore_barrier()
        pltpu.sync_copy(acc.at[pl.ds(wbase, APT)],
                        sums_out.at[pl.ds(g * ACC + wbase, APT)])
        plsc.subcore_barrier()

    pltpu.sync_copy(zrows, acc.at[pl.ds(wbase, APT)])
    plsc.subcore_barrier()
    hpt = ROWS // 2 // NS

    def cstep(k, _):
        c0 = 2 * k
        for cc, didx, ss, vals in ((c0, didx0, ss0, vals0),
                                   (c0 + 1, didx1, ss1, vals1)):
            rb = c * (ROWS // 2) + s * hpt + cc * CH

            @pl.when(jnp.logical_not(k == 0))
            def _():
                _drain(x8, vals, ss)
            pltpu.sync_copy(gdst.at[pl.ds(rb, CH)], didx)
            for j in range(CH):
                pltpu.async_copy(onesv, acc.at[didx.at[j]], ss, add=True)
        return 0

    lax.fori_loop(0, hpt // CH // 2, cstep, 0)
    _drain(x8, vals0, ss0)
    _drain(x8, vals1, ss1)
    plsc.subcore_barrier()
    pltpu.sync_copy(acc.at[pl.ds(wbase, APT)],
                    cnt_out.at[pl.ds(c * ACC + wbase, APT)])


def _r2_body(ytab, gsrc, gdst, zrows, t_out,
             acc, idx0, didx0, vals0, idx1, didx1, vals1,
             sg0, sg1, ss0, ss1):
    c = lax.axis_index("c")
    s = lax.axis_index("s")
    wbase = s * APT
    pltpu.sync_copy(zrows, acc.at[pl.ds(wbase, APT)])
    plsc.subcore_barrier()
    hpt = ROWS // 2 // NS

    def base(n):
        return c * (ROWS // 2) + s * hpt + n * CH

    _run_edge_pass(ytab, gsrc, gdst, base, base, None, hpt // CH, acc,
                   idx0, didx0, vals0, idx1, didx1, vals1,
                   sg0, sg1, ss0, ss1)
    plsc.subcore_barrier()
    pltpu.sync_copy(acc.at[pl.ds(wbase, APT)],
                    t_out.at[pl.ds(c * ACC + wbase, APT)])


_sc_mesh = plsc.VectorSubcoreMesh(core_axis_name="c", subcore_axis_name="s")
_sc_params = pltpu.CompilerParams(use_tc_tiling_on_sc=False)

_pipe_scratch = [
    pltpu.VMEM((CH, 128), i32),
    pltpu.VMEM((CH, 128), i32),
    pltpu.VMEM((CH * 128, 16), f32),
    pltpu.VMEM((CH, 128), i32),
    pltpu.VMEM((CH, 128), i32),
    pltpu.VMEM((CH * 128, 16), f32),
]
_pipe_sems = [pltpu.SemaphoreType.DMA] * 4

_r1_call = pl.kernel(
    _r1_body,
    compiler_params=_sc_params,
    out_type=(jax.ShapeDtypeStruct((8 * ACC, 16), f32),
              jax.ShapeDtypeStruct((2 * ACC, 16), f32)),
    mesh=_sc_mesh,
    scratch_types=[pltpu.VMEM_SHARED((ACC, 16), f32)] + _pipe_scratch
    + [pltpu.VMEM((128, 16), f32)] + _pipe_sems,
)

_r2_call = pl.kernel(
    _r2_body,
    compiler_params=_sc_params,
    out_type=jax.ShapeDtypeStruct((2 * ACC, 16), f32),
    mesh=_sc_mesh,
    scratch_types=[pltpu.VMEM_SHARED((ACC, 16), f32)] + _pipe_scratch
    + _pipe_sems,
)


BN = 2000
GRID = N // BN


def _lrelu(x):
    return jnp.where(x >= 0, x, NEG_SLOPE * x)


def _tc1_body(m_ref, fd_ref, w_ref, b_ref, o_ref, o8_ref):
    m = m_ref[...]
    d = fd_ref[...]
    w = w_ref[...]
    b = b_ref[...]
    a = _lrelu(jnp.dot(m, w, preferred_element_type=f32) + b)
    k = _lrelu(jnp.dot(d - m, w, preferred_element_type=f32) + b)
    full = jnp.concatenate([a, k], axis=1)
    o_ref[...] = full
    for g in range(8):
        o8_ref[g] = lax.slice(full, (0, 16 * g), (BN, 16 * (g + 1)))


def _tc2_body(x_ref, sa_ref, sb_ref, cnt_ref, wbig_ref, p2_ref,
              b1_ref, h_ref, y_ref):
    x = x_ref[...]
    sa = sa_ref[...]
    sb = sb_ref[...]
    cnt = cnt_ref[...]
    csum = jnp.maximum(cnt[0] + cnt[1], 1.0)
    m0 = jnp.concatenate([sa[g] for g in range(8)], axis=1) / csum[0]
    m1 = jnp.concatenate([sb[g] for g in range(8)], axis=1) / csum[1]
    big = jnp.concatenate([x, m0, m1], axis=1)
    h = jnp.dot(big, wbig_ref[...], preferred_element_type=f32) + b1_ref[...]
    h_ref[...] = h
    y01 = jnp.dot(h, p2_ref[...], preferred_element_type=f32)
    y_ref[...] = jnp.stack([y01[:, :16], y01[:, 16:]], axis=0)


def _tc3_body(h_ref, t_ref, cnt_ref, q_ref, bq_ref, o_ref):
    h = h_ref[...]
    t = t_ref[...]
    cnt = cnt_ref[...]
    csum = jnp.maximum(cnt[0] + cnt[1], 1.0)
    tsum = t[0] + t[1]
    agg = tsum[0] / csum[0] + tsum[1] / csum[1]
    o16 = jnp.dot(h, q_ref[...], preferred_element_type=f32) + bq_ref[...] + agg
    a = lax.slice(o16, (0, 0), (BN, OUT))
    k = lax.slice(o16, (0, OUT), (BN, 2 * OUT))
    o_ref[...] = a * k


def _block_diag(a):
    r, c = a.shape
    z = jnp.zeros((2 * r, 2 * c), f32)
    return z.at[:r, :c].set(a).at[r:, c:].set(a)


def _pad16(a):
    return jnp.pad(a, ((0, 0), (0, 16 - a.shape[1])))


@jax.jit
def _impl(mask_feature, feature, edge_index, edge_type,
          W_in, b_in, w1, root1, b1, w2, root2, b2, W_out, b_out):
    src = edge_index[0]
    dst = edge_index[1]
    npad = EP - E
    src_p = jnp.concatenate([src, jnp.zeros((npad,), i32)])
    gdst_p = jnp.concatenate([edge_type * N + dst,
                              jnp.full((npad,), GARBAGE, i32)])
    gsrc_p = jnp.concatenate([edge_type * N + src, jnp.zeros((npad,), i32)])
    src2 = src_p.reshape(ROWS, 128)
    gdst2 = gdst_p.reshape(ROWS, 128)
    gsrc2 = gsrc_p.reshape(ROWS, 128)

    x_fused, x8t = pl.pallas_call(
        _tc1_body,
        grid=(GRID,),
        in_specs=[
            pl.BlockSpec((BN, EMB), lambda i: (i, 0)),
            pl.BlockSpec((BN, EMB), lambda i: (i, 0)),
            pl.BlockSpec((EMB, HID), lambda i: (0, 0)),
            pl.BlockSpec((1, HID), lambda i: (0, 0)),
        ],
        out_specs=[
            pl.BlockSpec((BN, 2 * HID), lambda i: (i, 0)),
            pl.BlockSpec((8, BN, 16), lambda i: (0, i, 0)),
        ],
        out_shape=[
            jax.ShapeDtypeStruct((N, 2 * HID), f32),
            jax.ShapeDtypeStruct((8, N, 16), f32),
        ],
    )(mask_feature, feature, W_in, b_in[None, :])

    x8 = x8t.reshape(8 * N, 16)
    zrows = jnp.zeros((APT, 16), f32)
    sums, cnts = _r1_call(x8, src2, gdst2, zrows)
    sums3 = sums.reshape(8, ACC, 16)
    cnt4 = cnts.reshape(2, ACC, 16)[:, :2 * N, :1].reshape(2, 2, N, 1)

    wbig = jnp.concatenate([_block_diag(root1), _block_diag(w1[0]),
                            _block_diag(w1[1])], axis=0)
    p2 = jnp.concatenate([_pad16(_block_diag(w2[0] @ W_out)),
                          _pad16(_block_diag(w2[1] @ W_out))], axis=1)
    h1, y2 = pl.pallas_call(
        _tc2_body,
        grid=(GRID,),
        in_specs=[
            pl.BlockSpec((BN, 2 * HID), lambda i: (i, 0)),
            pl.BlockSpec((8, BN, 16), lambda i: (0, i, 0)),
            pl.BlockSpec((8, BN, 16), lambda i: (0, N // BN + i, 0)),
            pl.BlockSpec((2, 2, BN, 1), lambda i: (0, 0, i, 0)),
            pl.BlockSpec((3 * 2 * HID, 2 * HID), lambda i: (0, 0)),
            pl.BlockSpec((2 * HID, 32), lambda i: (0, 0)),
            pl.BlockSpec((1, 2 * HID), lambda i: (0, 0)),
        ],
        out_specs=[
            pl.BlockSpec((BN, 2 * HID), lambda i: (i, 0)),
            pl.BlockSpec((2, BN, 16), lambda i: (0, i, 0)),
        ],
        out_shape=[
            jax.ShapeDtypeStruct((N, 2 * HID), f32),
            jax.ShapeDtypeStruct((2, N, 16), f32),
        ],
    )(x_fused, sums3, sums3, cnt4, wbig, p2, jnp.tile(b1, 2)[None, :])

    ytab = y2.reshape(2 * N, 16)
    t_parts = _r2_call(ytab, gsrc2, gdst2, zrows)
    t4 = t_parts.reshape(2, ACC, 16)[:, :2 * N, :].reshape(2, 2, N, 16)

    q = _pad16(_block_diag(root2 @ W_out))
    bfin = b2 @ W_out + b_out
    bq = jnp.concatenate([bfin, bfin, jnp.zeros((16 - 2 * OUT,), f32)])[None, :]
    out = pl.pallas_call(
        _tc3_body,
        grid=(GRID,),
        in_specs=[
            pl.BlockSpec((BN, 2 * HID), lambda i: (i, 0)),
            pl.BlockSpec((2, 2, BN, 16), lambda i: (0, 0, i, 0)),
            pl.BlockSpec((2, 2, BN, 1), lambda i: (0, 0, i, 0)),
            pl.BlockSpec((2 * HID, 16), lambda i: (0, 0)),
            pl.BlockSpec((1, 16), lambda i: (0, 0)),
        ],
        out_specs=pl.BlockSpec((BN, OUT), lambda i: (i, 0)),
        out_shape=jax.ShapeDtypeStruct((N, OUT), f32),
    )(h1, t4, cnt4, q, bq)
    return out


def kernel(mask_feature, feature, edge_index, edge_type,
           W_in, b_in, w1, root1, b1, w2, root2, b2, W_out, b_out):
    return _impl(mask_feature, feature, edge_index, edge_type,
                 W_in, b_in, w1, root1, b1, w2, root2, b2, W_out, b_out)


# speedup vs baseline: 11.5540x; 1.2383x over previous
import functools

import jax
import jax.numpy as jnp
from jax import lax
from jax.experimental import pallas as pl
from jax.experimental.pallas import tpu as pltpu
from jax.experimental.pallas import tpu_sc as plsc

N = 50000
E = 800000
EMB = 128
HID = 64
OUT = 3
R = 2
NEG_SLOPE = 0.01

NC = 2
NS = 16
CH = 4

ROWS = 6400
EP = ROWS * 128
GARBAGE = 2 * N
ACC = 100096
APT = ACC // NS

f32 = jnp.float32
i32 = jnp.int32


def _fill_const(ref, nrows, val):
    def body(i, _):
        ref[i] = jnp.full((16,), val, f32)
        return 0
    lax.fori_loop(0, nrows, body, 0)


def _drain(table, buf, sem):
    pltpu.make_async_copy(table.at[pl.ds(0, CH * 128)], buf, sem).wait()


def _prefetch(table, src_idx, dst_idx, src_base, dst_base, src_off,
              idxv, didxv, valsv, sg, ss, first_round):
    @pl.when(jnp.logical_not(first_round))
    def _():
        _drain(table, valsv, ss)
    pltpu.sync_copy(src_idx.at[pl.ds(src_base, CH)], idxv)
    pltpu.sync_copy(dst_idx.at[pl.ds(dst_base, CH)], didxv)
    if src_off is not None:
        for jj in range(CH):
            for kk in range(8):
                idxv[jj, pl.ds(kk * 16, 16)] = idxv[jj, pl.ds(kk * 16, 16)] + src_off
    for j in range(CH):
        pltpu.async_copy(table.at[idxv.at[j]], valsv.at[pl.ds(j * 128, 128)], sg)


def _commit(table, didxv, valsv, acc, sg, ss):
    _drain(table, valsv, sg)
    for j in range(CH):
        pltpu.async_copy(valsv.at[pl.ds(j * 128, 128)], acc.at[didxv.at[j]],
                         ss, add=True)


def _run_edge_pass(table, src_idx, dst_idx, sbase_fn, dbase_fn, src_off,
                   nch, acc, idx0, didx0, vals0, idx1, didx1, vals1,
                   sg0, sg1, ss0, ss1):
    _prefetch(table, src_idx, dst_idx, sbase_fn(0), dbase_fn(0), src_off,
              idx0, didx0, vals0, sg0, ss0, jnp.bool_(True))

    def step(k, _):
        c0 = 2 * k
        c1 = c0 + 1
        _prefetch(table, src_idx, dst_idx, sbase_fn(c1), dbase_fn(c1),
                  src_off, idx1, didx1, vals1, sg1, ss1, k == 0)
        _commit(table, didx0, vals0, acc, sg0, ss0)

        @pl.when(c0 + 2 < nch)
        def _():
            _prefetch(table, src_idx, dst_idx, sbase_fn(c0 + 2),
                      dbase_fn(c0 + 2), src_off,
                      idx0, didx0, vals0, sg0, ss0, jnp.bool_(False))
        _commit(table, didx1, vals1, acc, sg1, ss1)
        return 0

    lax.fori_loop(0, nch // 2, step, 0)
    _drain(table, vals0, ss0)
    _drain(table, vals1, ss1)


def _r1_body(x8, src2, gdst, zrows, sums_out, cnt_out,
             acc, idx0, didx0, vals0, idx1, didx1, vals1, onesv,
             sg0, sg1, ss0, ss1):
    c = lax.axis_index("c")
    s = lax.axis_index("s")
    _fill_const(onesv, 128, 1.0)
    wbase = s * APT

    for p in range(4):
        g = c * 4 + p
        pltpu.sync_copy(zrows, acc.at[pl.ds(wbase, APT)])
        plsc.subcore_barrier()
        rpt = ROWS // NS
        _run_edge_pass(
            x8, src2, gdst,
            lambda n, rpt=rpt: s * rpt + n * CH,
            lambda n, rpt=rpt: s * rpt + n * CH,
            g, rpt // CH, acc,
            idx0, didx0, vals0, idx1, didx1, vals1, sg0, sg1, ss0, ss1)
        plsc.subcore_barrier()

        @pl.when(c == 0)
        def _(p=p):
            pltpu.sync_copy(acc.at[pl.ds(wbase, APT)],
                            sums_out.at[pl.ds(wbase, APT),
                                        pl.ds(16 * p, 16)])

        @pl.when(c == 1)
        def _(p=p):
            pltpu.sync_copy(acc.at[pl.ds(wbase, APT)],
                            sums_out.at[pl.ds(wbase, APT),
                                        pl.ds(16 * (4 + p), 16)])
        plsc.subcore_barrier()

    pltpu.sync_copy(zrows, acc.at[pl.ds(wbase, APT)])
    plsc.subcore_barrier()
    hpt = ROWS // 2 // NS

    def cstep(k, _):
        c0 = 2 * k
        for cc, didx, ss, vals in ((c0, didx0, ss0, vals0),
                                   (c0 + 1, didx1, ss1, vals1)):
            rb = c * (ROWS // 2) + s * hpt + cc * CH

            @pl.when(jnp.logical_not(k == 0))
            def _():
                _drain(x8, vals, ss)
            pltpu.sync_copy(gdst.at[pl.ds(rb, CH)], didx)
            for j in range(CH):
                pltpu.async_copy(onesv, acc.at[didx.at[j]], ss, add=True)
        return 0

    lax.fori_loop(0, hpt // CH // 2, cstep, 0)
    _drain(x8, vals0, ss0)
    _drain(x8, vals1, ss1)
    plsc.subcore_barrier()

    @pl.when(c == 0)
    def _():
        pltpu.sync_copy(acc.at[pl.ds(wbase, APT)],
                        cnt_out.at[pl.ds(wbase, APT), pl.ds(0, 16)])

    @pl.when(c == 1)
    def _():
        pltpu.sync_copy(acc.at[pl.ds(wbase, APT)],
                        cnt_out.at[pl.ds(wbase, APT), pl.ds(16, 16)])


def _r2_body(ytab, gsrc, gdst, zrows, t_out,
             acc, idx0, didx0, vals0, idx1, didx1, vals1,
             sg0, sg1, ss0, ss1):
    c = lax.axis_index("c")
    s = lax.axis_index("s")
    wbase = s * APT
    pltpu.sync_copy(zrows, acc.at[pl.ds(wbase, APT)])
    plsc.subcore_barrier()
    hpt = ROWS // 2 // NS

    def base(n):
        return c * (ROWS // 2) + s * hpt + n * CH

    _run_edge_pass(ytab, gsrc, gdst, base, base, None, hpt // CH, acc,
                   idx0, didx0, vals0, idx1, didx1, vals1,
                   sg0, sg1, ss0, ss1)
    plsc.subcore_barrier()

    @pl.when(c == 0)
    def _():
        pltpu.sync_copy(acc.at[pl.ds(wbase, APT)],
                        t_out.at[pl.ds(wbase, APT), pl.ds(0, 16)])

    @pl.when(c == 1)
    def _():
        pltpu.sync_copy(acc.at[pl.ds(wbase, APT)],
                        t_out.at[pl.ds(wbase, APT), pl.ds(16, 16)])


_sc_mesh = plsc.VectorSubcoreMesh(core_axis_name="c", subcore_axis_name="s")
_sc_params = pltpu.CompilerParams(use_tc_tiling_on_sc=False)

_pipe_scratch = [
    pltpu.VMEM((CH, 128), i32),
    pltpu.VMEM((CH, 128), i32),
    pltpu.VMEM((CH * 128, 16), f32),
    pltpu.VMEM((CH, 128), i32),
    pltpu.VMEM((CH, 128), i32),
    pltpu.VMEM((CH * 128, 16), f32),
]
_pipe_sems = [pltpu.SemaphoreType.DMA] * 4

_r1_call = pl.kernel(
    _r1_body,
    compiler_params=_sc_params,
    out_type=(jax.ShapeDtypeStruct((ACC, 128), f32),
              jax.ShapeDtypeStruct((ACC, 128), f32)),
    mesh=_sc_mesh,
    scratch_types=[pltpu.VMEM_SHARED((ACC, 16), f32)] + _pipe_scratch
    + [pltpu.VMEM((128, 16), f32)] + _pipe_sems,
)

_r2_call = pl.kernel(
    _r2_body,
    compiler_params=_sc_params,
    out_type=jax.ShapeDtypeStruct((ACC, 128), f32),
    mesh=_sc_mesh,
    scratch_types=[pltpu.VMEM_SHARED((ACC, 16), f32)] + _pipe_scratch
    + _pipe_sems,
)


BN = 2000
GRID = N // BN


def _lrelu(x):
    return jnp.where(x >= 0, x, NEG_SLOPE * x)


def _tc1_body(m_ref, fd_ref, w_ref, b_ref, o_ref):
    m = m_ref[...]
    d = fd_ref[...]
    w = w_ref[...]
    b = b_ref[...]
    a = _lrelu(jnp.dot(m, w, preferred_element_type=f32) + b)
    k = _lrelu(jnp.dot(d - m, w, preferred_element_type=f32) + b)
    o_ref[...] = jnp.concatenate([a, k], axis=1)


def _tc2_body(x_ref, sa_ref, sb_ref, ca_ref, cb_ref, wbig_ref, p2_ref,
              b1_ref, h_ref, y_ref):
    x = x_ref[...]
    c0 = jnp.maximum(ca_ref[:, 0:1] + ca_ref[:, 16:17], 1.0)
    c1 = jnp.maximum(cb_ref[:, 0:1] + cb_ref[:, 16:17], 1.0)
    m0 = sa_ref[...] / c0
    m1 = sb_ref[...] / c1
    big = jnp.concatenate([x, m0, m1], axis=1)
    h = jnp.dot(big, wbig_ref[...], preferred_element_type=f32) + b1_ref[...]
    h_ref[...] = h
    y01 = jnp.dot(h, p2_ref[...], preferred_element_type=f32)
    y_ref[...] = jnp.concatenate([y01, jnp.zeros((BN, 96), f32)], axis=1)


def _tc3_body(h_ref, ta_ref, tb_ref, ca_ref, cb_ref, q_ref, bq_ref, o_ref):
    h = h_ref[...]
    c0 = jnp.maximum(ca_ref[:, 0:1] + ca_ref[:, 16:17], 1.0)
    c1 = jnp.maximum(cb_ref[:, 0:1] + cb_ref[:, 16:17], 1.0)
    t0 = ta_ref[:, 0:16] + ta_ref[:, 16:32]
    t1 = tb_ref[:, 0:16] + tb_ref[:, 16:32]
    agg = t0 / c0 + t1 / c1
    o16 = jnp.dot(h, q_ref[...], preferred_element_type=f32) + bq_ref[...] + agg
    a = lax.slice(o16, (0, 0), (BN, OUT))
    k = lax.slice(o16, (0, OUT), (BN, 2 * OUT))
    o_ref[...] = a * k


def _block_diag(a):
    r, c = a.shape
    z = jnp.zeros((2 * r, 2 * c), f32)
    return z.at[:r, :c].set(a).at[r:, c:].set(a)


def _pad16(a):
    return jnp.pad(a, ((0, 0), (0, 16 - a.shape[1])))


@jax.jit
def _impl(mask_feature, feature, edge_index, edge_type,
          W_in, b_in, w1, root1, b1, w2, root2, b2, W_out, b_out):
    src = edge_index[0]
    dst = edge_index[1]
    npad = EP - E
    src_p = jnp.concatenate([src * 8, jnp.zeros((npad,), i32)])
    gdst_p = jnp.concatenate([edge_type * N + dst,
                              jnp.full((npad,), GARBAGE, i32)])
    gsrc_p = jnp.concatenate([src * 8 + edge_type, jnp.zeros((npad,), i32)])
    src2 = src_p.reshape(ROWS, 128)
    gdst2 = gdst_p.reshape(ROWS, 128)
    gsrc2 = gsrc_p.reshape(ROWS, 128)

    x_fused = pl.pallas_call(
        _tc1_body,
        grid=(GRID,),
        in_specs=[
            pl.BlockSpec((BN, EMB), lambda i: (i, 0)),
            pl.BlockSpec((BN, EMB), lambda i: (i, 0)),
            pl.BlockSpec((EMB, HID), lambda i: (0, 0)),
            pl.BlockSpec((1, HID), lambda i: (0, 0)),
        ],
        out_specs=pl.BlockSpec((BN, 2 * HID), lambda i: (i, 0)),
        out_shape=jax.ShapeDtypeStruct((N, 2 * HID), f32),
    )(mask_feature, feature, W_in, b_in[None, :])

    x8 = x_fused.reshape(8 * N, 16)
    zrows = jnp.zeros((APT, 16), f32)
    sums, cnts = _r1_call(x8, src2, gdst2, zrows)

    wbig = jnp.concatenate([_block_diag(root1), _block_diag(w1[0]),
                            _block_diag(w1[1])], axis=0)
    p2 = jnp.concatenate([_pad16(_block_diag(w2[0] @ W_out)),
                          _pad16(_block_diag(w2[1] @ W_out))], axis=1)
    h1, y2 = pl.pallas_call(
        _tc2_body,
        grid=(GRID,),
        in_specs=[
            pl.BlockSpec((BN, 2 * HID), lambda i: (i, 0)),
            pl.BlockSpec((BN, 128), lambda i: (i, 0)),
            pl.BlockSpec((BN, 128), lambda i: (N // BN + i, 0)),
            pl.BlockSpec((BN, 128), lambda i: (i, 0)),
            pl.BlockSpec((BN, 128), lambda i: (N // BN + i, 0)),
            pl.BlockSpec((3 * 2 * HID, 2 * HID), lambda i: (0, 0)),
            pl.BlockSpec((2 * HID, 32), lambda i: (0, 0)),
            pl.BlockSpec((1, 2 * HID), lambda i: (0, 0)),
        ],
        out_specs=[
            pl.BlockSpec((BN, 2 * HID), lambda i: (i, 0)),
            pl.BlockSpec((BN, 128), lambda i: (i, 0)),
        ],
        out_shape=[
            jax.ShapeDtypeStruct((N, 2 * HID), f32),
            jax.ShapeDtypeStruct((N, 128), f32),
        ],
    )(x_fused, sums, sums, cnts, cnts, wbig, p2, jnp.tile(b1, 2)[None, :])

    ytab = y2.reshape(8 * N, 16)
    t_parts = _r2_call(ytab, gsrc2, gdst2, zrows)

    q = _pad16(_block_diag(root2 @ W_out))
    bfin = b2 @ W_out + b_out
    bq = jnp.concatenate([bfin, bfin, jnp.zeros((16 - 2 * OUT,), f32)])[None, :]
    out = pl.pallas_call(
        _tc3_body,
        grid=(GRID,),
        in_specs=[
            pl.BlockSpec((BN, 2 * HID), lambda i: (i, 0)),
            pl.BlockSpec((BN, 128), lambda i: (i, 0)),
            pl.BlockSpec((BN, 128), lambda i: (N // BN + i, 0)),
            pl.BlockSpec((BN, 128), lambda i: (i, 0)),
            pl.BlockSpec((BN, 128), lambda i: (N // BN + i, 0)),
            pl.BlockSpec((2 * HID, 16), lambda i: (0, 0)),
            pl.BlockSpec((1, 16), lambda i: (0, 0)),
        ],
        out_specs=pl.BlockSpec((BN, OUT), lambda i: (i, 0)),
        out_shape=jax.ShapeDtypeStruct((N, OUT), f32),
    )(h1, t_parts, t_parts, cnts, cnts, q, bq)
    return out


def kernel(mask_feature, feature, edge_index, edge_type,
           W_in, b_in, w1, root1, b1, w2, root2, b2, W_out, b_out):
    return _impl(mask_feature, feature, edge_index, edge_type,
                 W_in, b_in, w1, root1, b1, w2, root2, b2, W_out, b_out)


# speedup vs baseline: 12.2656x; 1.0616x over previous
import functools

import jax
import jax.numpy as jnp
from jax import lax
from jax.experimental import pallas as pl
from jax.experimental.pallas import tpu as pltpu
from jax.experimental.pallas import tpu_sc as plsc

N = 50000
E = 800000
EMB = 128
HID = 64
OUT = 3
R = 2
NEG_SLOPE = 0.01

NC = 2
NS = 16
CH = 4

ROWS = 6400
EP = ROWS * 128
GARBAGE = 2 * N
ACC = 100096
APT = ACC // NS

f32 = jnp.float32
i32 = jnp.int32


def _fill_const(ref, nrows, val):
    def body(i, _):
        ref[i] = jnp.full((16,), val, f32)
        return 0
    lax.fori_loop(0, nrows, body, 0)


def _drain(table, buf, sem):
    pltpu.make_async_copy(table.at[pl.ds(0, CH * 128)], buf, sem).wait()


def _prefetch(table, src_idx, dst_idx, src_base, dst_base, src_off,
              idxv, didxv, valsv, sg, ss, first_round):
    @pl.when(jnp.logical_not(first_round))
    def _():
        _drain(table, valsv, ss)
    pltpu.sync_copy(src_idx.at[pl.ds(src_base, CH)], idxv)
    pltpu.sync_copy(dst_idx.at[pl.ds(dst_base, CH)], didxv)
    if src_off is not None:
        for jj in range(CH):
            for kk in range(8):
                idxv[jj, pl.ds(kk * 16, 16)] = idxv[jj, pl.ds(kk * 16, 16)] + src_off
    for j in range(CH):
        pltpu.async_copy(table.at[idxv.at[j]], valsv.at[pl.ds(j * 128, 128)], sg)


def _commit(table, didxv, valsv, acc, sg, ss):
    _drain(table, valsv, sg)
    for j in range(CH):
        pltpu.async_copy(valsv.at[pl.ds(j * 128, 128)], acc.at[didxv.at[j]],
                         ss, add=True)


def _run_edge_pass(table, src_idx, dst_idx, sbase_fn, dbase_fn, src_off,
                   nch, acc, idx0, didx0, vals0, idx1, didx1, vals1,
                   sg0, sg1, ss0, ss1):
    _prefetch(table, src_idx, dst_idx, sbase_fn(0), dbase_fn(0), src_off,
              idx0, didx0, vals0, sg0, ss0, jnp.bool_(True))

    def step(k, _):
        c0 = 2 * k
        c1 = c0 + 1
        _prefetch(table, src_idx, dst_idx, sbase_fn(c1), dbase_fn(c1),
                  src_off, idx1, didx1, vals1, sg1, ss1, k == 0)
        _commit(table, didx0, vals0, acc, sg0, ss0)

        @pl.when(c0 + 2 < nch)
        def _():
            _prefetch(table, src_idx, dst_idx, sbase_fn(c0 + 2),
                      dbase_fn(c0 + 2), src_off,
                      idx0, didx0, vals0, sg0, ss0, jnp.bool_(False))
        _commit(table, didx1, vals1, acc, sg1, ss1)
        return 0

    lax.fori_loop(0, nch // 2, step, 0)
    _drain(table, vals0, ss0)
    _drain(table, vals1, ss1)


def _r1_body(x8, src2, gdst, zrows, sums_out, cnt_out,
             acc, idx0, didx0, vals0, idx1, didx1, vals1, onesv,
             sg0, sg1, ss0, ss1):
    c = lax.axis_index("c")
    s = lax.axis_index("s")
    _fill_const(onesv, 128, 1.0)
    wbase = s * APT

    for p in range(4):
        g = c * 4 + p
        pltpu.sync_copy(zrows, acc.at[pl.ds(wbase, APT)])
        plsc.subcore_barrier()
        rpt = ROWS // NS
        _run_edge_pass(
            x8, src2, gdst,
            lambda n, rpt=rpt: s * rpt + n * CH,
            lambda n, rpt=rpt: s * rpt + n * CH,
            g * N, rpt // CH, acc,
            idx0, didx0, vals0, idx1, didx1, vals1, sg0, sg1, ss0, ss1)
        plsc.subcore_barrier()

        @pl.when(c == 0)
        def _(p=p):
            pltpu.sync_copy(acc.at[pl.ds(wbase, APT)],
                            sums_out.at[pl.ds(wbase, APT),
                                        pl.ds(16 * p, 16)])

        @pl.when(c == 1)
        def _(p=p):
            pltpu.sync_copy(acc.at[pl.ds(wbase, APT)],
                            sums_out.at[pl.ds(wbase, APT),
                                        pl.ds(16 * (4 + p), 16)])
        plsc.subcore_barrier()

    pltpu.sync_copy(zrows, acc.at[pl.ds(wbase, APT)])
    plsc.subcore_barrier()
    hpt = ROWS // 2 // NS

    def cstep(k, _):
        c0 = 2 * k
        for cc, didx, ss, vals in ((c0, didx0, ss0, vals0),
                                   (c0 + 1, didx1, ss1, vals1)):
            rb = c * (ROWS // 2) + s * hpt + cc * CH

            @pl.when(jnp.logical_not(k == 0))
            def _():
                _drain(x8, vals, ss)
            pltpu.sync_copy(gdst.at[pl.ds(rb, CH)], didx)
            for j in range(CH):
                pltpu.async_copy(onesv, acc.at[didx.at[j]], ss, add=True)
        return 0

    lax.fori_loop(0, hpt // CH // 2, cstep, 0)
    _drain(x8, vals0, ss0)
    _drain(x8, vals1, ss1)
    plsc.subcore_barrier()

    @pl.when(c == 0)
    def _():
        pltpu.sync_copy(acc.at[pl.ds(wbase, APT)],
                        cnt_out.at[pl.ds(wbase, APT), pl.ds(0, 16)])

    @pl.when(c == 1)
    def _():
        pltpu.sync_copy(acc.at[pl.ds(wbase, APT)],
                        cnt_out.at[pl.ds(wbase, APT), pl.ds(16, 16)])


def _r2_body(ytab, gsrc, gdst, zrows, t_out,
             acc, idx0, didx0, vals0, idx1, didx1, vals1,
             sg0, sg1, ss0, ss1):
    c = lax.axis_index("c")
    s = lax.axis_index("s")
    wbase = s * APT
    pltpu.sync_copy(zrows, acc.at[pl.ds(wbase, APT)])
    plsc.subcore_barrier()
    hpt = ROWS // 2 // NS

    def base(n):
        return c * (ROWS // 2) + s * hpt + n * CH

    _run_edge_pass(ytab, gsrc, gdst, base, base, None, hpt // CH, acc,
                   idx0, didx0, vals0, idx1, didx1, vals1,
                   sg0, sg1, ss0, ss1)
    plsc.subcore_barrier()

    @pl.when(c == 0)
    def _():
        pltpu.sync_copy(acc.at[pl.ds(wbase, APT)],
                        t_out.at[pl.ds(wbase, APT), pl.ds(0, 16)])

    @pl.when(c == 1)
    def _():
        pltpu.sync_copy(acc.at[pl.ds(wbase, APT)],
                        t_out.at[pl.ds(wbase, APT), pl.ds(16, 16)])


_sc_mesh = plsc.VectorSubcoreMesh(core_axis_name="c", subcore_axis_name="s")
_sc_params = pltpu.CompilerParams(use_tc_tiling_on_sc=False)

_pipe_scratch = [
    pltpu.VMEM((CH, 128), i32),
    pltpu.VMEM((CH, 128), i32),
    pltpu.VMEM((CH * 128, 16), f32),
    pltpu.VMEM((CH, 128), i32),
    pltpu.VMEM((CH, 128), i32),
    pltpu.VMEM((CH * 128, 16), f32),
]
_pipe_sems = [pltpu.SemaphoreType.DMA] * 4

_r1_call = pl.kernel(
    _r1_body,
    compiler_params=_sc_params,
    out_type=(jax.ShapeDtypeStruct((ACC, 128), f32),
              jax.ShapeDtypeStruct((ACC, 128), f32)),
    mesh=_sc_mesh,
    scratch_types=[pltpu.VMEM_SHARED((ACC, 16), f32)] + _pipe_scratch
    + [pltpu.VMEM((128, 16), f32)] + _pipe_sems,
)

_r2_call = pl.kernel(
    _r2_body,
    compiler_params=_sc_params,
    out_type=jax.ShapeDtypeStruct((ACC, 128), f32),
    mesh=_sc_mesh,
    scratch_types=[pltpu.VMEM_SHARED((ACC, 16), f32)] + _pipe_scratch
    + _pipe_sems,
)


BN = 2000
GRID = N // BN


def _lrelu(x):
    return jnp.where(x >= 0, x, NEG_SLOPE * x)


def _tc1_body(m_ref, fd_ref, w_ref, b_ref, o_ref):
    m = m_ref[...]
    d = fd_ref[...]
    w = w_ref[...]
    b = b_ref[...]
    a = _lrelu(jnp.dot(m, w, preferred_element_type=f32) + b)
    k = _lrelu(jnp.dot(d - m, w, preferred_element_type=f32) + b)
    o_ref[...] = jnp.concatenate([a, k], axis=1)


def _tc2_body(x_ref, sa_ref, sb_ref, ca_ref, cb_ref, wbig_ref, p2_ref,
              b1_ref, h_ref, y_ref):
    x = x_ref[...]
    c0 = jnp.maximum(ca_ref[:, 0:1] + ca_ref[:, 16:17], 1.0)
    c1 = jnp.maximum(cb_ref[:, 0:1] + cb_ref[:, 16:17], 1.0)
    m0 = sa_ref[...] / c0
    m1 = sb_ref[...] / c1
    big = jnp.concatenate([x, m0, m1], axis=1)
    h = jnp.dot(big, wbig_ref[...], preferred_element_type=f32) + b1_ref[...]
    h_ref[...] = h
    y01 = jnp.dot(h, p2_ref[...], preferred_element_type=f32)
    y_ref[...] = jnp.concatenate([y01, jnp.zeros((BN, 96), f32)], axis=1)


def _tc3_body(h_ref, ta_ref, tb_ref, ca_ref, cb_ref, q_ref, bq_ref, o_ref):
    h = h_ref[...]
    c0 = jnp.maximum(ca_ref[:, 0:1] + ca_ref[:, 16:17], 1.0)
    c1 = jnp.maximum(cb_ref[:, 0:1] + cb_ref[:, 16:17], 1.0)
    t0 = ta_ref[:, 0:16] + ta_ref[:, 16:32]
    t1 = tb_ref[:, 0:16] + tb_ref[:, 16:32]
    agg = t0 / c0 + t1 / c1
    o16 = jnp.dot(h, q_ref[...], preferred_element_type=f32) + bq_ref[...] + agg
    a = lax.slice(o16, (0, 0), (BN, OUT))
    k = lax.slice(o16, (0, OUT), (BN, 2 * OUT))
    o_ref[...] = a * k


def _block_diag(a):
    r, c = a.shape
    z = jnp.zeros((2 * r, 2 * c), f32)
    return z.at[:r, :c].set(a).at[r:, c:].set(a)


def _pad16(a):
    return jnp.pad(a, ((0, 0), (0, 16 - a.shape[1])))


@jax.jit
def _impl(mask_feature, feature, edge_index, edge_type,
          W_in, b_in, w1, root1, b1, w2, root2, b2, W_out, b_out):
    src = edge_index[0]
    dst = edge_index[1]
    npad = EP - E
    src_p = jnp.concatenate([src, jnp.zeros((npad,), i32)])
    gdst_p = jnp.concatenate([edge_type * N + dst,
                              jnp.full((npad,), GARBAGE, i32)])
    gsrc_p = jnp.concatenate([src * 8 + edge_type, jnp.zeros((npad,), i32)])
    src2 = src_p.reshape(ROWS, 128)
    gdst2 = gdst_p.reshape(ROWS, 128)
    gsrc2 = gsrc_p.reshape(ROWS, 128)

    x_fused = pl.pallas_call(
        _tc1_body,
        grid=(GRID,),
        in_specs=[
            pl.BlockSpec((BN, EMB), lambda i: (i, 0)),
            pl.BlockSpec((BN, EMB), lambda i: (i, 0)),
            pl.BlockSpec((EMB, HID), lambda i: (0, 0)),
            pl.BlockSpec((1, HID), lambda i: (0, 0)),
        ],
        out_specs=pl.BlockSpec((BN, 2 * HID), lambda i: (i, 0)),
        out_shape=jax.ShapeDtypeStruct((N, 2 * HID), f32),
    )(mask_feature, feature, W_in, b_in[None, :])

    x8 = (x_fused.reshape(N, 8, 16).transpose(1, 0, 2)
          .reshape(N, 128).reshape(8 * N, 16))
    zrows = jnp.zeros((APT, 16), f32)
    sums, cnts = _r1_call(x8, src2, gdst2, zrows)

    wbig = jnp.concatenate([_block_diag(root1), _block_diag(w1[0]),
                            _block_diag(w1[1])], axis=0)
    p2 = jnp.concatenate([_pad16(_block_diag(w2[0] @ W_out)),
                          _pad16(_block_diag(w2[1] @ W_out))], axis=1)
    h1, y2 = pl.pallas_call(
        _tc2_body,
        grid=(GRID,),
        in_specs=[
            pl.BlockSpec((BN, 2 * HID), lambda i: (i, 0)),
            pl.BlockSpec((BN, 128), lambda i: (i, 0)),
            pl.BlockSpec((BN, 128), lambda i: (N // BN + i, 0)),
            pl.BlockSpec((BN, 128), lambda i: (i, 0)),
            pl.BlockSpec((BN, 128), lambda i: (N // BN + i, 0)),
            pl.BlockSpec((3 * 2 * HID, 2 * HID), lambda i: (0, 0)),
            pl.BlockSpec((2 * HID, 32), lambda i: (0, 0)),
            pl.BlockSpec((1, 2 * HID), lambda i: (0, 0)),
        ],
        out_specs=[
            pl.BlockSpec((BN, 2 * HID), lambda i: (i, 0)),
            pl.BlockSpec((BN, 128), lambda i: (i, 0)),
        ],
        out_shape=[
            jax.ShapeDtypeStruct((N, 2 * HID), f32),
            jax.ShapeDtypeStruct((N, 128), f32),
        ],
    )(x_fused, sums, sums, cnts, cnts, wbig, p2, jnp.tile(b1, 2)[None, :])

    ytab = y2.reshape(8 * N, 16)
    t_parts = _r2_call(ytab, gsrc2, gdst2, zrows)

    q = _pad16(_block_diag(root2 @ W_out))
    bfin = b2 @ W_out + b_out
    bq = jnp.concatenate([bfin, bfin, jnp.zeros((16 - 2 * OUT,), f32)])[None, :]
    out = pl.pallas_call(
        _tc3_body,
        grid=(GRID,),
        in_specs=[
            pl.BlockSpec((BN, 2 * HID), lambda i: (i, 0)),
            pl.BlockSpec((BN, 128), lambda i: (i, 0)),
            pl.BlockSpec((BN, 128), lambda i: (N // BN + i, 0)),
            pl.BlockSpec((BN, 128), lambda i: (i, 0)),
            pl.BlockSpec((BN, 128), lambda i: (N // BN + i, 0)),
            pl.BlockSpec((2 * HID, 16), lambda i: (0, 0)),
            pl.BlockSpec((1, 16), lambda i: (0, 0)),
        ],
        out_specs=pl.BlockSpec((BN, OUT), lambda i: (i, 0)),
        out_shape=jax.ShapeDtypeStruct((N, OUT), f32),
    )(h1, t_parts, t_parts, cnts, cnts, q, bq)
    return out


def kernel(mask_feature, feature, edge_index, edge_type,
           W_in, b_in, w1, root1, b1, w2, root2, b2, W_out, b_out):
    return _impl(mask_feature, feature, edge_index, edge_type,
                 W_in, b_in, w1, root1, b1, w2, root2, b2, W_out, b_out)
